# baseline (speedup 1.0000x reference)
import functools

import jax
import jax.numpy as jnp
from jax import lax
from jax.experimental import pallas as pl
from jax.experimental.pallas import tpu as pltpu
from jax.experimental.pallas import tpu_sc as plsc

_EMBED = 64
_LANES = 16
_NC = 2
_NS = 16
_NW = _NC * _NS
_CHUNK = 128
_NBUF = 8
_LEAD = 6
_TBLK = 512


@functools.lru_cache(maxsize=None)
def _make_idx_transpose(nb: int, nl: int):
    def body(x_ref, o_ref):
        o_ref[...] = jnp.swapaxes(x_ref[...], 0, 1).reshape(nl, nb // _CHUNK, _CHUNK)

    return pl.pallas_call(
        body,
        out_shape=jax.ShapeDtypeStruct((nl, nb // _CHUNK, _CHUNK), jnp.int32),
    )


@functools.lru_cache(maxsize=None)
def _make_emb_kernel(nl: int, nb: int):
    nlt = nl // _NBUF
    assert nb == _NW * _CHUNK and nl % _NBUF == 0 and nlt >= 3
    mesh = plsc.VectorSubcoreMesh(core_axis_name="c", subcore_axis_name="s")

    @functools.partial(
        pl.kernel,
        out_type=jax.ShapeDtypeStruct((nb, nl, _EMBED), jnp.float32),
        mesh=mesh,
        scratch_types=[
            pltpu.VMEM((nl, 1, _CHUNK), jnp.int32),
            pltpu.VMEM((_NBUF, _CHUNK, _EMBED), jnp.float32),
            pltpu.SemaphoreType.DMA((_NBUF,)),
            pltpu.SemaphoreType.DMA((_NBUF,)),
        ],
        compiler_params=pltpu.CompilerParams(use_tc_tiling_on_sc=False),
    )
    def emb(idx_hbm, table_hbm, out_hbm, idx_v, rows_v, gsem, ssem):
        wid = lax.axis_index("s") * _NC + lax.axis_index("c")
        col = wid * _CHUNK
        pltpu.sync_copy(idx_hbm.at[:, pl.ds(wid, 1)], idx_v)

        def gather_issue(l, b):
            pltpu.async_copy(
                table_hbm.at[idx_v.at[l, 0]], rows_v.at[b], gsem.at[b]
            )

        def gather_wait(b):
            pltpu.make_async_copy(
                table_hbm.at[pl.ds(0, _CHUNK)], rows_v.at[b], gsem.at[b]
            ).wait()

        def scatter_issue(l, b):
            pltpu.async_copy(
                rows_v.at[b],
                out_hbm.at[pl.ds(col, _CHUNK), l],
                ssem.at[b],
            )

        def scatter_wait(b):
            pltpu.make_async_copy(
                rows_v.at[b],
                out_hbm.at[pl.ds(0, _CHUNK), 0],
                ssem.at[b],
            ).wait()

        def scale(b):
            @pl.loop(0, _CHUNK, unroll=8)
            def _(i):
                for j in range(_EMBED // _LANES):
                    sl = pl.ds(j * _LANES, _LANES)
                    rows_v[b, i, sl] = rows_v[b, i, sl] * 8.0

        for ls in range(_LEAD):
            gather_issue(ls, ls)

        for ls in range(_NBUF):
            gather_wait(ls)
            scale(ls)
            scatter_issue(ls, ls)
            if ls >= 2:
                scatter_wait((ls - 2) % _NBUF)
            gather_issue(ls + _LEAD, (ls + _LEAD) % _NBUF)

        @pl.loop(1, nlt - 1)
        def _(lt):
            l0 = lt * _NBUF
            for ls in range(_NBUF):
                gather_wait(ls)
                scale(ls)
                scatter_issue(l0 + ls, ls)
                scatter_wait((ls + _LEAD) % _NBUF)
                gather_issue(l0 + ls + _LEAD, (ls + _LEAD) % _NBUF)

        for ls in range(_NBUF):
            gather_wait(ls)
            scale(ls)
            scatter_issue(nl - _NBUF + ls, ls)
            if ls + _LEAD < _NBUF:
                scatter_wait(ls + _LEAD)
                gather_issue(nl - _NBUF + ls + _LEAD, ls + _LEAD)

        for b in range(_NBUF):
            scatter_wait(b)

    return emb


def kernel(x, input_embedding_table):
    nb, nl = x.shape
    idx3 = _make_idx_transpose(nb, nl)(x)
    return _make_emb_kernel(nl, nb)(idx3, input_embedding_table)
